# baseline (speedup 1.0000x reference)
import functools

import numpy as np
import jax
import jax.numpy as jnp
from jax import lax
from jax.experimental import pallas as pl
from jax.experimental.pallas import tpu as pltpu
from jax.experimental.pallas import tpu_sc as plsc

_N = 10000
_F = 128
_NC, _NS = 2, 16
_NW = _NC * _NS
_E_TOT = 320000 + _N
_E_PAD = 344064
_RPT = 632
_NPAD = _NS * _RPT

_f32 = jnp.float32
_i32 = jnp.int32


def _make_edge_kernel(H, C, K, nchunk, split):
    HC = H * C
    WS = HC + 16
    mesh = plsc.VectorSubcoreMesh(core_axis_name="c", subcore_axis_name="s")

    def body(tsrc, tdst, eidx_h, out_h,
             c0, c1, c2, c3, c4, c5, c6, c7,
             r0, r1, r2, r3, a0, a1, a2, a3, zrow, acc,
             si0, si1, si2, si3, si4, si5, si6, si7,
             sr0, sr1, sr2, sr3, sa0, sa1, sa2, sa3,
             ss0, ss1, ss2, ss3):
        cidxs = (c0, c1, c2, c3, c4, c5, c6, c7)
        rowss = (r0, r1, r2, r3)
        arowss = (a0, a1, a2, a3)
        semis = (si0, si1, si2, si3, si4, si5, si6, si7)
        semrs = (sr0, sr1, sr2, sr3)
        semas = (sa0, sa1, sa2, sa3)
        semss = (ss0, ss1, ss2, ss3)

        cid = lax.axis_index("c")
        sid = lax.axis_index("s")
        g = sid if split else cid * _NS + sid
        tsrc_c = tsrc.at[cid] if split else tsrc
        tdst_c = tdst.at[cid] if split else tdst
        iota = lax.iota(_i32, 16)
        zv = iota.astype(_f32) * 0.0
        LAST = nchunk - 1

        def z_body(r, c):
            for buf in rowss + (zrow,):
                for col in range(WS // 16):
                    buf[r, pl.ds(col * 16, 16)] = zv
            return c
        lax.fori_loop(0, K, z_body, 0)

        nfull, nrem = _RPT // K, _RPT % K
        for b in range(nfull):
            pltpu.sync_copy(zrow, acc.at[pl.ds(sid * _RPT + b * K, K)])
        if nrem:
            pltpu.sync_copy(zrow.at[pl.ds(0, nrem)],
                            acc.at[pl.ds(sid * _RPT + nfull * K, nrem)])
        pltpu.sync_copy(eidx_h.at[g, 0], cidxs[0])
        plsc.subcore_barrier()

        def clampj(j):
            return jnp.minimum(j, LAST)

        def start_idx(j, m8):
            pltpu.async_copy(eidx_h.at[g, clampj(j)], cidxs[m8], semis[m8])

        def wait_idx(j, m8):
            pltpu.make_async_copy(
                eidx_h.at[g, clampj(j)], cidxs[m8], semis[m8]).wait()

        def start_gather(m8, m4):
            pltpu.async_copy(tsrc_c.at[cidxs[m8].at[0]], rowss[m4], semrs[m4])
            pltpu.async_copy(tdst_c.at[cidxs[m8].at[1]], arowss[m4], semas[m4])

        def wait_gather(m8, m4):
            pltpu.make_async_copy(
                tsrc_c.at[cidxs[m8].at[0]], rowss[m4], semrs[m4]).wait()
            pltpu.make_async_copy(
                tdst_c.at[cidxs[m8].at[1]], arowss[m4], semas[m4]).wait()

        def start_scatter(m8, m4):
            pltpu.async_copy(rowss[m4], acc.at[cidxs[m8].at[1]],
                             semss[m4], add=True)

        def wait_scatter(m8, m4):
            pltpu.make_async_copy(rowss[m4], acc.at[cidxs[m8].at[1]],
                                  semss[m4]).wait()

        hmask = iota < H

        def compute(m4):
            rows = rowss[m4]
            arows = arowss[m4]

            @plsc.parallel_loop(0, K, 1, unroll=8)
            def e_body(e):
                av = rows[e, pl.ds(HC, 16)]
                bv = arows[e, pl.ds(0, 16)]
                a = av + bv
                a = jnp.maximum(a, 0.2 * a)
                ex = jnp.exp(a)
                ex = jnp.where(hmask, ex, 0.0)
                rows[e, pl.ds(HC, 16)] = ex
                for h in range(H):
                    scv = ex.at[iota * 0 + h].get(mode="promise_in_bounds")
                    off = h * C
                    rows[e, pl.ds(off, 16)] = rows[e, pl.ds(off, 16)] * scv

        for m4 in (2, 3):
            pltpu.async_copy(zrow, acc.at[cidxs[0].at[1]], semss[m4], add=True)
        pltpu.sync_copy(eidx_h.at[g, 1], cidxs[1])
        start_gather(0, 0)
        start_gather(1, 1)
        start_idx(2, 2)

        def step(i, t):
            wait_idx(i + 2, (t + 2) % 8)
            wait_scatter((t + 6) % 8, (t + 2) % 4)
            start_gather((t + 2) % 8, (t + 2) % 4)
            start_idx(i + 3, (t + 3) % 8)
            wait_gather(t % 8, t % 4)
            compute(t % 4)
            start_scatter(t % 8, t % 4)

        def loop_body(jj, c):
            j0 = jj * 8
            for t in range(8):
                step(j0 + t, t)
            return c

        lax.fori_loop(0, nchunk // 8, loop_body, 0)
        wait_scatter(6, 2)
        wait_scatter(7, 3)
        wait_gather(0, 0)
        wait_gather(1, 1)
        wait_idx(LAST, 2)

        plsc.subcore_barrier()
        pltpu.sync_copy(acc.at[pl.ds(sid * _RPT, _RPT)],
                        out_h.at[cid, pl.ds(sid * _RPT, _RPT)])

    dma = pltpu.SemaphoreType.DMA
    return pl.kernel(
        body,
        out_type=jax.ShapeDtypeStruct((_NC, _NPAD, WS), _f32),
        mesh=mesh,
        scratch_types=(
            [pltpu.VMEM((2, K), _i32)] * 8
            + [pltpu.VMEM((K, WS), _f32)] * 4
            + [pltpu.VMEM((K, 16), _f32)] * 4
            + [pltpu.VMEM((K, WS), _f32)]
            + [pltpu.VMEM_SHARED((_NPAD, WS), _f32)]
            + [dma] * 20
        ),
        compiler_params=pltpu.CompilerParams(use_tc_tiling_on_sc=False),
    )


_K1, _NCH1 = 128, 168
_K2, _NCH2 = 96, 112
_edge1 = _make_edge_kernel(4, 16, _K1, _NCH1, split=True)
_edge2 = _make_edge_kernel(1, 16, _K2, _NCH2, split=False)


_BM = 632
_BMF = 1000


def _mm1_body(x_ref, w_ref, o1_ref, o2_ref):
    t = jnp.dot(x_ref[...], w_ref[0], preferred_element_type=_f32)
    o1_ref[0] = t[:, :80]
    o2_ref[0] = t[:, 80:]


def _tc_l1(xp, w1full):
    return pl.pallas_call(
        _mm1_body,
        grid=(2, _NPAD // _BM),
        in_specs=[pl.BlockSpec((_BM, 128), lambda c, i: (i, 0)),
                  pl.BlockSpec((1, 128, 96), lambda c, i: (c, 0, 0))],
        out_specs=[pl.BlockSpec((1, _BM, 80), lambda c, i: (c, i, 0)),
                   pl.BlockSpec((1, _BM, 16), lambda c, i: (c, i, 0))],
        out_shape=[jax.ShapeDtypeStruct((2, _NPAD, 80), _f32),
                   jax.ShapeDtypeStruct((2, _NPAD, 16), _f32)],
    )(xp, w1full)


def _mid_body(acc_ref, b_ref, w_ref, r_ref, o1_ref, o2_ref):
    s0 = acc_ref[0]
    s1 = acc_ref[1]
    num = jnp.concatenate([s0[:, :64], s1[:, :64]], axis=1)
    den8 = jnp.concatenate([s0[:, 64:68], s1[:, 64:68]], axis=1)
    den = jnp.dot(den8, r_ref[...], preferred_element_type=_f32)
    gv = num / den + b_ref[...]
    gv = jnp.where(gv > 0, gv, jnp.exp(gv) - 1.0)
    t = jnp.dot(gv, w_ref[...], preferred_element_type=_f32)
    o1_ref[...] = t[:, :32]
    o2_ref[...] = t[:, 32:]


def _tc_mid(acc1, b1, w2full, r8):
    return pl.pallas_call(
        _mid_body,
        grid=(_NPAD // _BM,),
        in_specs=[pl.BlockSpec((_NC, _BM, 80), lambda i: (0, i, 0)),
                  pl.BlockSpec((1, 128), lambda i: (0, 0)),
                  pl.BlockSpec((128, 48), lambda i: (0, 0)),
                  pl.BlockSpec((8, 128), lambda i: (0, 0))],
        out_specs=[pl.BlockSpec((_BM, 32), lambda i: (i, 0)),
                   pl.BlockSpec((_BM, 16), lambda i: (i, 0))],
        out_shape=[jax.ShapeDtypeStruct((_NPAD, 32), _f32),
                   jax.ShapeDtypeStruct((_NPAD, 16), _f32)],
    )(acc1, b1, w2full, r8)


def _fin_body(acc_ref, b_ref, o_ref):
    s = acc_ref[0] + acc_ref[1]
    num = s[:, :16]
    den = jnp.broadcast_to(s[:, 16:17], (_BMF, 16))
    o = num / den + b_ref[...]
    o_ref[...] = jnp.where(o > 0, o, jnp.exp(o) - 1.0)


def _tc_fin(acc2, b2):
    return pl.pallas_call(
        _fin_body,
        grid=(_N // _BMF,),
        in_specs=[pl.BlockSpec((_NC, _BMF, 32), lambda i: (0, i, 0)),
                  pl.BlockSpec((1, 16), lambda i: (0, 0))],
        out_specs=pl.BlockSpec((_BMF, 16), lambda i: (i, 0)),
        out_shape=jax.ShapeDtypeStruct((_N, 16), _f32),
    )(acc2, b2)


def _build_wext(W, a_s, a_d, H, C, width):
    HC = H * C
    rows_idx = jnp.arange(HC)
    A_s = jnp.zeros((HC, H), _f32).at[rows_idx, rows_idx // C].set(a_s.reshape(-1))
    A_d = jnp.zeros((HC, H), _f32).at[rows_idx, rows_idx // C].set(a_d.reshape(-1))
    wext = jnp.concatenate([W, W @ A_s, W @ A_d], axis=1)
    return jnp.pad(wext, ((0, 0), (0, width - wext.shape[1])))


def kernel(x, edge_index, W1, a_src1, a_dst1, b1, W2, a_src2, a_dst2, b2):
    ridx = jnp.arange(128)
    A1s = jnp.zeros((128, 8), _f32).at[ridx, ridx // 16].set(a_src1.reshape(-1))
    A1d = jnp.zeros((128, 8), _f32).at[ridx, ridx // 16].set(a_dst1.reshape(-1))
    W1As, W1Ad = W1 @ A1s, W1 @ A1d
    z12 = jnp.zeros((128, 12), _f32)
    w1cores = [jnp.concatenate(
        [W1[:, 64 * c:64 * c + 64], W1As[:, 4 * c:4 * c + 4], z12,
         W1Ad[:, 4 * c:4 * c + 4], z12], axis=1) for c in range(2)]
    w1full = jnp.stack(w1cores)

    w2ext = _build_wext(W2, a_src2, a_dst2, 1, 16, 32)
    w2full = jnp.concatenate(
        [w2ext, W2 @ a_dst2.T, jnp.zeros((128, 15), _f32)], axis=1)
    r8 = jnp.kron(jnp.eye(8, dtype=_f32), jnp.ones((1, 16), _f32))

    loops = jnp.arange(_N, dtype=_i32)
    src = jnp.concatenate([edge_index[0].astype(_i32), loops])
    dst = jnp.concatenate([edge_index[1].astype(_i32), loops])
    pad_n = _E_PAD - _E_TOT
    src = jnp.pad(src, (0, pad_n), constant_values=_N)
    dst = jnp.pad(dst, (0, pad_n), constant_values=_N)
    eidx1 = jnp.stack([src.reshape(_NS, _NCH1, _K1),
                       dst.reshape(_NS, _NCH1, _K1)], axis=2)
    eidx2 = jnp.stack([src.reshape(_NW, _NCH2, _K2),
                       dst.reshape(_NW, _NCH2, _K2)], axis=2)

    xp = jnp.pad(x, ((0, _NPAD - _N), (0, 0)))

    t1p, d1 = _tc_l1(xp, w1full)
    acc1 = _edge1(t1p, d1, eidx1)

    t2p, d2 = _tc_mid(acc1, b1.reshape(1, 128), w2full, r8)
    acc2 = _edge2(t2p, d2, eidx2)

    return _tc_fin(acc2, b2.reshape(1, 16))


# speedup vs baseline: 1.9225x; 1.9225x over previous
import functools

import numpy as np
import jax
import jax.numpy as jnp
from jax import lax
from jax.experimental import pallas as pl
from jax.experimental.pallas import tpu as pltpu
from jax.experimental.pallas import tpu_sc as plsc

_N = 10000
_F = 128
_NC, _NS = 2, 16
_NW = _NC * _NS
_E_TOT = 320000 + _N
_E_PAD = 331776
_RPT = 632
_NPAD = _NS * _RPT

_f32 = jnp.float32
_i32 = jnp.int32


def _make_edge_kernel(H, C, K, nchunk, split):
    HC = H * C
    WS = HC + 16
    mesh = plsc.VectorSubcoreMesh(core_axis_name="c", subcore_axis_name="s")

    def body(tsrc, tdst, eidx_h, out_h,
             cidx0, cidx1, rows0, rows1, arows0, arows1, acc,
             sem_i0, sem_i1, sem_s0, sem_s1, sem_a0, sem_a1):
        cid = lax.axis_index("c")
        sid = lax.axis_index("s")
        g = sid if split else cid * _NS + sid
        tsrc_c = tsrc.at[cid] if split else tsrc
        tdst_c = tdst.at[cid] if split else tdst
        iota = lax.iota(_i32, 16)

        zv = iota.astype(_f32) * 0.0

        def z_body(r, c):
            for col in range(WS // 16):
                rows0[r, pl.ds(col * 16, 16)] = zv
            return c
        lax.fori_loop(0, K, z_body, 0)
        nfull, nrem = _RPT // K, _RPT % K
        for b in range(nfull):
            pltpu.sync_copy(rows0, acc.at[pl.ds(sid * _RPT + b * K, K)])
        if nrem:
            pltpu.sync_copy(rows0.at[pl.ds(0, nrem)],
                            acc.at[pl.ds(sid * _RPT + nfull * K, nrem)])
        plsc.subcore_barrier()

        bufs = ((cidx0, rows0, arows0, sem_i0, sem_s0, sem_a0),
                (cidx1, rows1, arows1, sem_i1, sem_s1, sem_a1))

        def start_idx(j, b):
            cidx, _, _, si, _, _ = bufs[b]
            pltpu.async_copy(eidx_h.at[g, j], cidx, si)

        def wait_idx(j, b):
            cidx, _, _, si, _, _ = bufs[b]
            pltpu.make_async_copy(eidx_h.at[g, j], cidx, si).wait()

        def start_gather(b):
            cidx, rows, arows, _, ss, sa = bufs[b]
            pltpu.async_copy(tsrc_c.at[cidx.at[0]], rows, ss)
            pltpu.async_copy(tdst_c.at[cidx.at[1]], arows, sa)

        def wait_gather(b):
            cidx, rows, arows, _, ss, sa = bufs[b]
            pltpu.make_async_copy(tsrc_c.at[cidx.at[0]], rows, ss).wait()
            pltpu.make_async_copy(tdst_c.at[cidx.at[1]], arows, sa).wait()

        def compute_scatter(b):
            cidx, rows, arows, _, _, _ = bufs[b]

            @plsc.parallel_loop(0, K, 1, unroll=4)
            def e_body(e):
                av = rows[e, pl.ds(HC, 16)]
                bv = arows[e, pl.ds(0, 16)]
                a = av + bv
                a = jnp.maximum(a, 0.2 * a)
                ex = jnp.exp(a)
                ex = jnp.where(iota < H, ex, 0.0)
                rows[e, pl.ds(HC, 16)] = ex
                for h in range(H):
                    scv = ex.at[iota * 0 + h].get(mode="promise_in_bounds")
                    off = h * C
                    rows[e, pl.ds(off, 16)] = rows[e, pl.ds(off, 16)] * scv

            pltpu.sync_copy(rows, acc.at[cidx.at[1]], add=True)

        pltpu.sync_copy(eidx_h.at[g, 0], cidx0)
        start_gather(0)
        start_idx(1, 1)

        def step(j, b):
            jn = jnp.minimum(j + 1, nchunk - 1)
            wait_idx(jn, 1 - b)
            start_gather(1 - b)
            wait_gather(b)
            compute_scatter(b)
            start_idx(jnp.minimum(j + 2, nchunk - 1), b)

        def loop_body(jj, c):
            step(jj * 2, 0)
            step(jj * 2 + 1, 1)
            return c

        lax.fori_loop(0, nchunk // 2, loop_body, 0)
        wait_gather(0)
        wait_idx(nchunk - 1, 1)

        plsc.subcore_barrier()
        pltpu.sync_copy(acc.at[pl.ds(sid * _RPT, _RPT)],
                        out_h.at[cid, pl.ds(sid * _RPT, _RPT)])

    return pl.kernel(
        body,
        out_type=jax.ShapeDtypeStruct((_NC, _NPAD, WS), _f32),
        mesh=mesh,
        scratch_types=[
            pltpu.VMEM((2, K), _i32),
            pltpu.VMEM((2, K), _i32),
            pltpu.VMEM((K, WS), _f32),
            pltpu.VMEM((K, WS), _f32),
            pltpu.VMEM((K, 16), _f32),
            pltpu.VMEM((K, 16), _f32),
            pltpu.VMEM_SHARED((_NPAD, WS), _f32),
            pltpu.SemaphoreType.DMA,
            pltpu.SemaphoreType.DMA,
            pltpu.SemaphoreType.DMA,
            pltpu.SemaphoreType.DMA,
            pltpu.SemaphoreType.DMA,
            pltpu.SemaphoreType.DMA,
        ],
        compiler_params=pltpu.CompilerParams(use_tc_tiling_on_sc=False),
    )


_K1, _NCH1 = 96, 108
_K2, _NCH2 = 96, 108
_edge1 = _make_edge_kernel(8, 16, _K1, _NCH1, split=False)
_edge2 = _make_edge_kernel(1, 16, _K2, _NCH2, split=False)


_BM = 632
_BMF = 1000


def _mm_body(x_ref, w_ref, o1_ref, o2_ref):
    t = jnp.dot(x_ref[...], w_ref[...], preferred_element_type=_f32)
    n1 = o1_ref.shape[1]
    o1_ref[...] = t[:, :n1]
    o2_ref[...] = t[:, n1:]


def _tc_matmul2(x, w, n1):
    m, k = x.shape
    n = w.shape[1]
    return pl.pallas_call(
        _mm_body,
        grid=(m // _BM,),
        in_specs=[pl.BlockSpec((_BM, k), lambda i: (i, 0)),
                  pl.BlockSpec((k, n), lambda i: (0, 0))],
        out_specs=[pl.BlockSpec((_BM, n1), lambda i: (i, 0)),
                   pl.BlockSpec((_BM, n - n1), lambda i: (i, 0))],
        out_shape=[jax.ShapeDtypeStruct((m, n1), _f32),
                   jax.ShapeDtypeStruct((m, n - n1), _f32)],
    )(x, w)


def _mid_body(acc_ref, b_ref, w_ref, r_ref, o1_ref, o2_ref):
    s = acc_ref[0] + acc_ref[1]
    num = s[:, :128]
    den = jnp.dot(s[:, 128:136], r_ref[...], preferred_element_type=_f32)
    gv = num / den + b_ref[...]
    gv = jnp.where(gv > 0, gv, jnp.exp(gv) - 1.0)
    t = jnp.dot(gv, w_ref[...], preferred_element_type=_f32)
    o1_ref[...] = t[:, :32]
    o2_ref[...] = t[:, 32:]


def _tc_mid(acc1, b1, w2full, r8):
    return pl.pallas_call(
        _mid_body,
        grid=(_NPAD // _BM,),
        in_specs=[pl.BlockSpec((_NC, _BM, 144), lambda i: (0, i, 0)),
                  pl.BlockSpec((1, 128), lambda i: (0, 0)),
                  pl.BlockSpec((128, 48), lambda i: (0, 0)),
                  pl.BlockSpec((8, 128), lambda i: (0, 0))],
        out_specs=[pl.BlockSpec((_BM, 32), lambda i: (i, 0)),
                   pl.BlockSpec((_BM, 16), lambda i: (i, 0))],
        out_shape=[jax.ShapeDtypeStruct((_NPAD, 32), _f32),
                   jax.ShapeDtypeStruct((_NPAD, 16), _f32)],
    )(acc1, b1, w2full, r8)


def _fin_body(acc_ref, b_ref, o_ref):
    s = acc_ref[0] + acc_ref[1]
    num = s[:, :16]
    den = jnp.broadcast_to(s[:, 16:17], (_BMF, 16))
    o = num / den + b_ref[...]
    o_ref[...] = jnp.where(o > 0, o, jnp.exp(o) - 1.0)


def _tc_fin(acc2, b2):
    return pl.pallas_call(
        _fin_body,
        grid=(_N // _BMF,),
        in_specs=[pl.BlockSpec((_NC, _BMF, 32), lambda i: (0, i, 0)),
                  pl.BlockSpec((1, 16), lambda i: (0, 0))],
        out_specs=pl.BlockSpec((_BMF, 16), lambda i: (i, 0)),
        out_shape=jax.ShapeDtypeStruct((_N, 16), _f32),
    )(acc2, b2)


def _build_wext(W, a_s, a_d, H, C, width):
    HC = H * C
    rows_idx = jnp.arange(HC)
    A_s = jnp.zeros((HC, H), _f32).at[rows_idx, rows_idx // C].set(a_s.reshape(-1))
    A_d = jnp.zeros((HC, H), _f32).at[rows_idx, rows_idx // C].set(a_d.reshape(-1))
    wext = jnp.concatenate([W, W @ A_s, W @ A_d], axis=1)
    return jnp.pad(wext, ((0, 0), (0, width - wext.shape[1])))


def kernel(x, edge_index, W1, a_src1, a_dst1, b1, W2, a_src2, a_dst2, b2):
    w1ext = _build_wext(W1, a_src1, a_dst1, 8, 16, 144)
    ridx = jnp.arange(128)
    A1d = jnp.zeros((128, 8), _f32).at[ridx, ridx // 16].set(a_dst1.reshape(-1))
    w1full = jnp.concatenate(
        [w1ext, W1 @ A1d, jnp.zeros((128, 8), _f32)], axis=1)
    w2ext = _build_wext(W2, a_src2, a_dst2, 1, 16, 32)
    w2full = jnp.concatenate(
        [w2ext, W2 @ a_dst2.T, jnp.zeros((128, 15), _f32)], axis=1)
    r8 = jnp.kron(jnp.eye(8, dtype=_f32), jnp.ones((1, 16), _f32))

    loops = jnp.arange(_N, dtype=_i32)
    src = jnp.concatenate([edge_index[0].astype(_i32), loops])
    dst = jnp.concatenate([edge_index[1].astype(_i32), loops])
    pad_n = _E_PAD - _E_TOT
    src = jnp.pad(src, (0, pad_n), constant_values=_N)
    dst = jnp.pad(dst, (0, pad_n), constant_values=_N)
    eidx = jnp.stack([src.reshape(_NW, _NCH1, _K1),
                      dst.reshape(_NW, _NCH1, _K1)], axis=2)

    xp = jnp.pad(x, ((0, _NPAD - _N), (0, 0)))

    t1p, d1 = _tc_matmul2(xp, w1full, 144)
    acc1 = _edge1(t1p, d1, eidx)

    t2p, d2 = _tc_mid(acc1, b1.reshape(1, 128), w2full, r8)
    acc2 = _edge2(t2p, d2, eidx)

    return _tc_fin(acc2, b2.reshape(1, 16))
